# baseline (speedup 1.0000x reference)
import functools

import jax
import jax.numpy as jnp
from jax import lax
from jax.experimental import pallas as pl
from jax.experimental.pallas import tpu as pltpu
from jax.experimental.pallas import tpu_sc as plsc

VOCAB = 1000000
EMBED = 64
BATCH = 4096
SEQ = 50

NC = 2
NS = 16
NW = NC * NS
ROWS_PER_W = BATCH // NW
TOK_PER_W = ROWS_PER_W * SEQ
CHUNK = 128
NCHUNK = TOK_PER_W // CHUNK


def _pooled_sum_sc(tok2d, pattern, zeros, table):
    mesh = plsc.VectorSubcoreMesh(core_axis_name="c", subcore_axis_name="s")

    @functools.partial(
        pl.kernel,
        out_type=jax.ShapeDtypeStruct((BATCH, EMBED), jnp.float32),
        mesh=mesh,
        compiler_params=pltpu.CompilerParams(use_tc_tiling_on_sc=False),
        scratch_types=[
            pltpu.VMEM((NCHUNK, CHUNK), jnp.int32),
            pltpu.VMEM((NCHUNK, CHUNK), jnp.int32),
            pltpu.VMEM_SHARED((NS * ROWS_PER_W, EMBED), jnp.float32),
            pltpu.VMEM((CHUNK, EMBED), jnp.float32),
            pltpu.VMEM((CHUNK, EMBED), jnp.float32),
            pltpu.SemaphoreType.DMA,
            pltpu.SemaphoreType.DMA,
        ],
    )
    def k(tok_hbm, pat_hbm, zer_hbm, table_hbm, out_hbm,
          ids_v, pat_v, acc_sh, buf0, buf1, sem0, sem1):
        sid = lax.axis_index("s")
        wid = sid * NC + lax.axis_index("c")
        pltpu.sync_copy(tok_hbm.at[wid], ids_v)
        pltpu.sync_copy(pat_hbm.at[sid], pat_v)
        pltpu.sync_copy(zer_hbm, acc_sh.at[pl.ds(sid * ROWS_PER_W, ROWS_PER_W)])

        @pl.loop(0, NCHUNK, step=2)
        def _(j):
            c0 = pltpu.async_copy(table_hbm.at[ids_v.at[j]], buf0, sem0)
            c1 = pltpu.async_copy(table_hbm.at[ids_v.at[j + 1]], buf1, sem1)
            c0.wait()
            pltpu.sync_copy(buf0, acc_sh.at[pat_v.at[j]], add=True)
            c1.wait()
            pltpu.sync_copy(buf1, acc_sh.at[pat_v.at[j + 1]], add=True)

        pltpu.sync_copy(acc_sh.at[pl.ds(sid * ROWS_PER_W, ROWS_PER_W)],
                        out_hbm.at[pl.ds(wid * ROWS_PER_W, ROWS_PER_W)])

    return k(tok2d, pattern, zeros, table)


def _mlp_tc(pooled_sum, W1, b1, W2, b2):
    nblk = 8
    blk = BATCH // nblk

    def body(p_ref, w1_ref, b1_ref, w2_ref, b2_ref, o_ref):
        x = p_ref[...] * (1.0 / SEQ)
        h = lax.dot_general(x, w1_ref[...], (((1,), (1,)), ((), ())),
                            precision=lax.Precision.HIGHEST,
                            preferred_element_type=jnp.float32)
        h = h + b1_ref[...]
        h = h * jax.nn.sigmoid(h)
        o = lax.dot_general(h, w2_ref[...], (((1,), (1,)), ((), ())),
                            precision=lax.Precision.HIGHEST,
                            preferred_element_type=jnp.float32)
        o_ref[...] = o + b2_ref[...]

    return pl.pallas_call(
        body,
        grid=(nblk,),
        in_specs=[
            pl.BlockSpec((blk, EMBED), lambda i: (i, 0)),
            pl.BlockSpec((EMBED, EMBED), lambda i: (0, 0)),
            pl.BlockSpec((1, EMBED), lambda i: (0, 0)),
            pl.BlockSpec((EMBED, EMBED), lambda i: (0, 0)),
            pl.BlockSpec((1, EMBED), lambda i: (0, 0)),
        ],
        out_specs=pl.BlockSpec((blk, EMBED), lambda i: (i, 0)),
        out_shape=jax.ShapeDtypeStruct((BATCH, EMBED), jnp.float32),
    )(pooled_sum, W1, b1.reshape(1, EMBED), W2, b2.reshape(1, EMBED))


def kernel(token_ids, table, W1, b1, W2, b2):
    tok2d = token_ids.reshape(NW, NCHUNK, CHUNK).astype(jnp.int32)
    local = (jnp.arange(TOK_PER_W, dtype=jnp.int32) // SEQ).reshape(1, NCHUNK, CHUNK)
    base = (jnp.arange(NS, dtype=jnp.int32) * ROWS_PER_W).reshape(NS, 1, 1)
    pattern = local + base
    zeros = jnp.zeros((ROWS_PER_W, EMBED), jnp.float32)
    pooled_sum = _pooled_sum_sc(tok2d, pattern, zeros, table)
    return _mlp_tc(pooled_sum, W1, b1, W2, b2)
